# baseline (speedup 1.0000x reference)
import functools

import jax
import jax.numpy as jnp
from jax import lax
from jax.experimental import pallas as pl
from jax.experimental.pallas import tpu as pltpu
from jax.experimental.pallas import tpu_sc as plsc

N = 10000
E = 320000
D = 128

NC, NS = 2, 16
NW = NC * NS
EPW = E // NW
KG = 80
NCH = EPW // KG
NP = 10240
RPT = NP // NS


def _mesh():
    return plsc.VectorSubcoreMesh(
        core_axis_name="c", subcore_axis_name="s", num_cores=NC, num_subcores=NS)


def _silu(x):
    return x * jax.nn.sigmoid(x)


def _tc_pq(feat, W1a, W1b, b_e1):
    BN = 1000

    def body(f_ref, wa, wb, b1, p_ref, q_ref):
        f = f_ref[...]
        p_ref[...] = jnp.dot(f, wa[...], preferred_element_type=jnp.float32) + b1[...]
        q_ref[...] = jnp.dot(f, wb[...], preferred_element_type=jnp.float32)

    return pl.pallas_call(
        body,
        grid=(N // BN,),
        in_specs=[
            pl.BlockSpec((BN, D), lambda i: (i, 0)),
            pl.BlockSpec((D, D), lambda i: (0, 0)),
            pl.BlockSpec((D, D), lambda i: (0, 0)),
            pl.BlockSpec((1, D), lambda i: (0, 0)),
        ],
        out_specs=[
            pl.BlockSpec((BN, D), lambda i: (i, 0)),
            pl.BlockSpec((BN, D), lambda i: (i, 0)),
        ],
        out_shape=(
            jax.ShapeDtypeStruct((N, D), jnp.float32),
            jax.ShapeDtypeStruct((N, D), jnp.float32),
        ),
    )(feat, W1a, W1b, b_e1)


def _sc_gather_combine(P, Q, copad, src, dst):
    @functools.partial(
        pl.kernel,
        out_type=(
            jax.ShapeDtypeStruct((E, D), jnp.float32),
            jax.ShapeDtypeStruct((E, 16), jnp.float32),
        ),
        mesh=_mesh(),
        scratch_types=[
            pltpu.VMEM((EPW,), jnp.int32),
            pltpu.VMEM((EPW,), jnp.int32),
            pltpu.VMEM((KG, D), jnp.float32),
            pltpu.VMEM((KG, D), jnp.float32),
            pltpu.VMEM((KG, 16), jnp.float32),
            pltpu.VMEM((KG, 16), jnp.float32),
            pltpu.VMEM((KG, D), jnp.float32),
            pltpu.VMEM((KG, D), jnp.float32),
            pltpu.VMEM((KG, 16), jnp.float32),
            pltpu.VMEM((KG, 16), jnp.float32),
            pltpu.SemaphoreType.DMA,
            pltpu.SemaphoreType.DMA,
            pltpu.SemaphoreType.DMA,
            pltpu.SemaphoreType.DMA,
        ],
        compiler_params=pltpu.CompilerParams(use_tc_tiling_on_sc=False),
    )
    def k(p_hbm, q_hbm, co_hbm, src_hbm, dst_hbm, z_hbm, dx_hbm,
          idx_s, idx_d, pv0, qv0, xsv0, xdv0, pv1, qv1, xsv1, xdv1,
          gsem0, gsem1, wsem0, wsem1):
        c = lax.axis_index("c")
        s = lax.axis_index("s")
        base = (s * NC + c) * EPW
        pltpu.sync_copy(src_hbm.at[pl.ds(base, EPW)], idx_s)
        pltpu.sync_copy(dst_hbm.at[pl.ds(base, EPW)], idx_d)

        bufs = ((pv0, qv0, xsv0, xdv0, gsem0, wsem0),
                (pv1, qv1, xsv1, xdv1, gsem1, wsem1))

        def issue_gathers(slot, b):
            pv, qv, xsv, xdv, gsem, _ = bufs[b]
            isl = idx_s.at[pl.ds(slot * KG, KG)]
            idl = idx_d.at[pl.ds(slot * KG, KG)]
            pltpu.async_copy(p_hbm.at[isl], pv, gsem)
            pltpu.async_copy(q_hbm.at[idl], qv, gsem)
            pltpu.async_copy(co_hbm.at[isl], xsv, gsem)
            pltpu.async_copy(co_hbm.at[idl], xdv, gsem)

        def wait_gathers(b):
            pv, qv, xsv, xdv, gsem, _ = bufs[b]
            pltpu.make_async_copy(p_hbm.at[idx_s.at[pl.ds(0, KG)]], pv, gsem).wait()
            pltpu.make_async_copy(q_hbm.at[idx_d.at[pl.ds(0, KG)]], qv, gsem).wait()
            pltpu.make_async_copy(co_hbm.at[idx_s.at[pl.ds(0, KG)]], xsv, gsem).wait()
            pltpu.make_async_copy(co_hbm.at[idx_d.at[pl.ds(0, KG)]], xdv, gsem).wait()

        def compute(b):
            pv, qv, xsv, xdv, _, _ = bufs[b]

            def edge(e, carry):
                for j in range(8):
                    sl = pl.ds(16 * j, 16)
                    pv[e, sl] = pv[e, sl] + qv[e, sl]
                xsv[e, :] = xsv[e, :] - xdv[e, :]
                return carry

            lax.fori_loop(0, KG, edge, 0, unroll=2)

        def issue_wb(slot, b):
            pv, _, xsv, _, _, wsem = bufs[b]
            off = base + slot * KG
            pltpu.async_copy(pv, z_hbm.at[pl.ds(off, KG)], wsem)
            pltpu.async_copy(xsv, dx_hbm.at[pl.ds(off, KG)], wsem)

        def wait_wb(b):
            pv, _, xsv, _, _, wsem = bufs[b]
            pltpu.make_async_copy(pv, z_hbm.at[pl.ds(base, KG)], wsem).wait()
            pltpu.make_async_copy(xsv, dx_hbm.at[pl.ds(base, KG)], wsem).wait()

        issue_gathers(0, 0)

        def pair(i, carry):
            slot0 = 2 * i
            wait_gathers(0)

            @pl.when(i >= 1)
            def _():
                wait_wb(1)

            issue_gathers(slot0 + 1, 1)
            compute(0)
            issue_wb(slot0, 0)
            wait_gathers(1)
            wait_wb(0)
            issue_gathers(slot0 + 2, 0)
            compute(1)
            issue_wb(slot0 + 1, 1)
            return carry

        lax.fori_loop(0, (NCH - 1) // 2, pair, 0)
        wait_gathers(0)
        compute(0)
        wait_wb(1)
        issue_wb(NCH - 1, 0)
        wait_wb(0)

    return k(P, Q, copad, src, dst)


def _tc_edge(Z0, DX, w1c, W_e2, b_e2, W_c1, b_c1, W_c2, b_c2):
    BE = 2000

    def body(z_ref, dx_ref, w1c_ref, w2, b2, wc1, bc1, wc2, bc2, he_ref, xe_ref):
        dx = dx_ref[...]
        dist2 = jnp.sum(dx * dx, axis=-1, keepdims=True)
        z = z_ref[...] + dist2 * w1c_ref[...]
        a1 = _silu(z)
        he = _silu(jnp.dot(a1, w2[...], preferred_element_type=jnp.float32) + b2[...])
        t = _silu(jnp.dot(he, wc1[...], preferred_element_type=jnp.float32) + bc1[...])
        coef = jnp.dot(t, wc2[...], preferred_element_type=jnp.float32) + bc2[...]
        he_ref[...] = he
        xe_ref[...] = dx * coef

    return pl.pallas_call(
        body,
        grid=(E // BE,),
        in_specs=[
            pl.BlockSpec((BE, D), lambda i: (i, 0)),
            pl.BlockSpec((BE, 16), lambda i: (i, 0)),
            pl.BlockSpec((1, D), lambda i: (0, 0)),
            pl.BlockSpec((D, D), lambda i: (0, 0)),
            pl.BlockSpec((1, D), lambda i: (0, 0)),
            pl.BlockSpec((D, D), lambda i: (0, 0)),
            pl.BlockSpec((1, D), lambda i: (0, 0)),
            pl.BlockSpec((D, 1), lambda i: (0, 0)),
            pl.BlockSpec((1, 1), lambda i: (0, 0)),
        ],
        out_specs=[
            pl.BlockSpec((BE, D), lambda i: (i, 0)),
            pl.BlockSpec((BE, 16), lambda i: (i, 0)),
        ],
        out_shape=(
            jax.ShapeDtypeStruct((E, D), jnp.float32),
            jax.ShapeDtypeStruct((E, 16), jnp.float32),
        ),
    )(Z0, DX, w1c, W_e2, b_e2, W_c1, b_c1, W_c2, b_c2)


def _sc_scatter_h(HE, dst, zero_h):
    @functools.partial(
        pl.kernel,
        out_type=jax.ShapeDtypeStruct((NC, NP, D), jnp.float32),
        mesh=_mesh(),
        scratch_types=[
            pltpu.VMEM((KG,), jnp.int32),
            pltpu.VMEM((KG,), jnp.int32),
            pltpu.VMEM((KG, D), jnp.float32),
            pltpu.VMEM((KG, D), jnp.float32),
            pltpu.VMEM_SHARED((NP, D), jnp.float32),
            pltpu.SemaphoreType.DMA,
            pltpu.SemaphoreType.DMA,
        ],
        compiler_params=pltpu.CompilerParams(use_tc_tiling_on_sc=True),
    )
    def k(he_hbm, dst_hbm, zh_hbm, outh_hbm,
          idx_d0, idx_d1, hev0, hev1, acc_h, lsem0, lsem1):
        c = lax.axis_index("c")
        s = lax.axis_index("s")
        rs = s * RPT
        pltpu.sync_copy(zh_hbm, acc_h.at[pl.ds(rs, RPT)])

        w = c * NS + s
        base = w * EPW
        plsc.subcore_barrier()

        bufs = ((hev0, idx_d0, lsem0), (hev1, idx_d1, lsem1))

        def issue_loads(slot, b):
            hev, idx_db, lsem = bufs[b]
            off = base + slot * KG
            pltpu.async_copy(he_hbm.at[pl.ds(off, KG)], hev, lsem)
            pltpu.async_copy(dst_hbm.at[pl.ds(off, KG)], idx_db, lsem)

        def wait_loads(b):
            hev, idx_db, lsem = bufs[b]
            pltpu.make_async_copy(he_hbm.at[pl.ds(base, KG)], hev, lsem).wait()
            pltpu.make_async_copy(dst_hbm.at[pl.ds(base, KG)], idx_db, lsem).wait()

        def scatter(b):
            hev, idx_db, _ = bufs[b]
            pltpu.sync_copy(hev, acc_h.at[idx_db], add=True)

        issue_loads(0, 0)

        def pair(i, carry):
            slot0 = 2 * i
            wait_loads(0)
            issue_loads(slot0 + 1, 1)
            scatter(0)
            wait_loads(1)
            issue_loads(slot0 + 2, 0)
            scatter(1)
            return carry

        lax.fori_loop(0, (NCH - 1) // 2, pair, 0)
        wait_loads(0)
        scatter(0)

        plsc.subcore_barrier()
        pltpu.sync_copy(acc_h.at[pl.ds(rs, RPT)], outh_hbm.at[c, pl.ds(rs, RPT)])

    return k(HE, dst, zero_h)


def _sc_scatter_x(XE, dst, zero_x):
    @functools.partial(
        pl.kernel,
        out_type=jax.ShapeDtypeStruct((NC, NP, 16), jnp.float32),
        mesh=_mesh(),
        scratch_types=[
            pltpu.VMEM((KG,), jnp.int32),
            pltpu.VMEM((KG,), jnp.int32),
            pltpu.VMEM((KG, 16), jnp.float32),
            pltpu.VMEM((KG, 16), jnp.float32),
            pltpu.VMEM_SHARED((NP, 16), jnp.float32),
            pltpu.SemaphoreType.DMA,
            pltpu.SemaphoreType.DMA,
        ],
        compiler_params=pltpu.CompilerParams(use_tc_tiling_on_sc=False),
    )
    def k(xe_hbm, dst_hbm, zx_hbm, outx_hbm,
          idx_d0, idx_d1, xev0, xev1, acc_x, lsem0, lsem1):
        c = lax.axis_index("c")
        s = lax.axis_index("s")
        rs = s * RPT
        pltpu.sync_copy(zx_hbm, acc_x.at[pl.ds(rs, RPT)])

        w = c * NS + s
        base = w * EPW
        plsc.subcore_barrier()

        bufs = ((xev0, idx_d0, lsem0), (xev1, idx_d1, lsem1))

        def issue_loads(slot, b):
            xev, idx_db, lsem = bufs[b]
            off = base + slot * KG
            pltpu.async_copy(xe_hbm.at[pl.ds(off, KG)], xev, lsem)
            pltpu.async_copy(dst_hbm.at[pl.ds(off, KG)], idx_db, lsem)

        def wait_loads(b):
            xev, idx_db, lsem = bufs[b]
            pltpu.make_async_copy(xe_hbm.at[pl.ds(base, KG)], xev, lsem).wait()
            pltpu.make_async_copy(dst_hbm.at[pl.ds(base, KG)], idx_db, lsem).wait()

        def scatter(b):
            xev, idx_db, _ = bufs[b]
            pltpu.sync_copy(xev, acc_x.at[idx_db], add=True)

        issue_loads(0, 0)

        def pair(i, carry):
            slot0 = 2 * i
            wait_loads(0)
            issue_loads(slot0 + 1, 1)
            scatter(0)
            wait_loads(1)
            issue_loads(slot0 + 2, 0)
            scatter(1)
            return carry

        lax.fori_loop(0, (NCH - 1) // 2, pair, 0)
        wait_loads(0)
        scatter(0)

        plsc.subcore_barrier()
        pltpu.sync_copy(acc_x.at[pl.ds(rs, RPT)], outx_hbm.at[c, pl.ds(rs, RPT)])

    return k(XE, dst, zero_x)


def _tc_node(feat, coordinate, PH, PX, Wn1a, Wn1b, b_n1, W_n2, b_n2):
    BN = 1000

    def body(f_ref, co_ref, ph, px, wa, wb, b1, w2, b2, h_ref, x_ref):
        hagg = ph[0] + ph[1]
        a = _silu(jnp.dot(f_ref[...], wa[...], preferred_element_type=jnp.float32)
                  + jnp.dot(hagg, wb[...], preferred_element_type=jnp.float32)
                  + b1[...])
        h_ref[...] = jnp.dot(a, w2[...], preferred_element_type=jnp.float32) + b2[...]
        xagg = px[0] + px[1]
        x_ref[...] = co_ref[...] + xagg[:, :3]

    return pl.pallas_call(
        body,
        grid=(N // BN,),
        in_specs=[
            pl.BlockSpec((BN, D), lambda i: (i, 0)),
            pl.BlockSpec((BN, 3), lambda i: (i, 0)),
            pl.BlockSpec((NC, BN, D), lambda i: (0, i, 0)),
            pl.BlockSpec((NC, BN, 16), lambda i: (0, i, 0)),
            pl.BlockSpec((D, D), lambda i: (0, 0)),
            pl.BlockSpec((D, D), lambda i: (0, 0)),
            pl.BlockSpec((1, D), lambda i: (0, 0)),
            pl.BlockSpec((D, D), lambda i: (0, 0)),
            pl.BlockSpec((1, D), lambda i: (0, 0)),
        ],
        out_specs=[
            pl.BlockSpec((BN, D), lambda i: (i, 0)),
            pl.BlockSpec((BN, 3), lambda i: (i, 0)),
        ],
        out_shape=(
            jax.ShapeDtypeStruct((N, D), jnp.float32),
            jax.ShapeDtypeStruct((N, 3), jnp.float32),
        ),
    )(feat, coordinate, PH, PX, Wn1a, Wn1b, b_n1, W_n2, b_n2)


def kernel(feat, coordinate, edge_index, W_e1, b_e1, W_e2, b_e2,
           W_c1, b_c1, W_c2, b_c2, W_n1, b_n1, W_n2, b_n2):
    src = edge_index[0]
    dst = edge_index[1]
    W1a = W_e1[:D]
    W1b = W_e1[D:2 * D]
    w1c = W_e1[2 * D].reshape(1, D)
    copad = jnp.pad(coordinate, ((0, 0), (0, 13)))

    P, Q = _tc_pq(feat, W1a, W1b, b_e1.reshape(1, D))
    Z0, DXp = _sc_gather_combine(P, Q, copad, src, dst)
    HE, XEp = _tc_edge(Z0, DXp, w1c, W_e2, b_e2.reshape(1, D),
                       W_c1, b_c1.reshape(1, D), W_c2, b_c2.reshape(1, 1))
    zero_h = jnp.zeros((RPT, D), jnp.float32)
    zero_x = jnp.zeros((RPT, 16), jnp.float32)
    PH = _sc_scatter_h(HE, dst, zero_h)
    PX = _sc_scatter_x(XEp, dst, zero_x)

    h_out, x_out = _tc_node(feat, coordinate, PH, PX,
                            W_n1[:D], W_n1[D:], b_n1.reshape(1, D),
                            W_n2, b_n2.reshape(1, D))
    return (h_out, x_out)


# speedup vs baseline: 1.0341x; 1.0341x over previous
import functools

import jax
import jax.numpy as jnp
from jax import lax
from jax.experimental import pallas as pl
from jax.experimental.pallas import tpu as pltpu
from jax.experimental.pallas import tpu_sc as plsc

N = 10000
E = 320000
D = 128

NC, NS = 2, 16
NW = NC * NS
EPW = E // NW
KG = 80
NCH = EPW // KG
NP = 10240
RPT = NP // NS


def _mesh():
    return plsc.VectorSubcoreMesh(
        core_axis_name="c", subcore_axis_name="s", num_cores=NC, num_subcores=NS)


def _silu(x):
    return x * jax.nn.sigmoid(x)


def _tc_pq(feat, W1a, W1b, b_e1):
    BN = 1000

    def body(f_ref, wa, wb, b1, p_ref, q_ref):
        f = f_ref[...]
        p_ref[...] = jnp.dot(f, wa[...], preferred_element_type=jnp.float32) + b1[...]
        q_ref[...] = jnp.dot(f, wb[...], preferred_element_type=jnp.float32)

    return pl.pallas_call(
        body,
        grid=(N // BN,),
        in_specs=[
            pl.BlockSpec((BN, D), lambda i: (i, 0)),
            pl.BlockSpec((D, D), lambda i: (0, 0)),
            pl.BlockSpec((D, D), lambda i: (0, 0)),
            pl.BlockSpec((1, D), lambda i: (0, 0)),
        ],
        out_specs=[
            pl.BlockSpec((BN, D), lambda i: (i, 0)),
            pl.BlockSpec((BN, D), lambda i: (i, 0)),
        ],
        out_shape=(
            jax.ShapeDtypeStruct((N, D), jnp.float32),
            jax.ShapeDtypeStruct((N, D), jnp.float32),
        ),
    )(feat, W1a, W1b, b_e1)


def _sc_gather_combine(P, Q, copad, src, dst):
    @functools.partial(
        pl.kernel,
        out_type=(
            jax.ShapeDtypeStruct((E, D), jnp.float32),
            jax.ShapeDtypeStruct((E, 16), jnp.float32),
        ),
        mesh=_mesh(),
        scratch_types=[
            pltpu.VMEM((EPW,), jnp.int32),
            pltpu.VMEM((EPW,), jnp.int32),
            pltpu.VMEM((KG, D), jnp.float32),
            pltpu.VMEM((KG, D), jnp.float32),
            pltpu.VMEM((KG, 16), jnp.float32),
            pltpu.VMEM((KG, 16), jnp.float32),
            pltpu.VMEM((KG, D), jnp.float32),
            pltpu.VMEM((KG, D), jnp.float32),
            pltpu.VMEM((KG, 16), jnp.float32),
            pltpu.VMEM((KG, 16), jnp.float32),
            pltpu.SemaphoreType.DMA,
            pltpu.SemaphoreType.DMA,
            pltpu.SemaphoreType.DMA,
            pltpu.SemaphoreType.DMA,
        ],
        compiler_params=pltpu.CompilerParams(use_tc_tiling_on_sc=False),
    )
    def k(p_hbm, q_hbm, co_hbm, src_hbm, dst_hbm, z_hbm, dx_hbm,
          idx_s, idx_d, pv0, qv0, xsv0, xdv0, pv1, qv1, xsv1, xdv1,
          gsem0, gsem1, wsem0, wsem1):
        c = lax.axis_index("c")
        s = lax.axis_index("s")
        base = (s * NC + c) * EPW
        pltpu.sync_copy(src_hbm.at[pl.ds(base, EPW)], idx_s)
        pltpu.sync_copy(dst_hbm.at[pl.ds(base, EPW)], idx_d)

        bufs = ((pv0, qv0, xsv0, xdv0, gsem0, wsem0),
                (pv1, qv1, xsv1, xdv1, gsem1, wsem1))

        def issue_gathers(slot, b):
            pv, qv, xsv, xdv, gsem, _ = bufs[b]
            isl = idx_s.at[pl.ds(slot * KG, KG)]
            idl = idx_d.at[pl.ds(slot * KG, KG)]
            pltpu.async_copy(p_hbm.at[isl], pv, gsem)
            pltpu.async_copy(q_hbm.at[idl], qv, gsem)
            pltpu.async_copy(co_hbm.at[isl], xsv, gsem)
            pltpu.async_copy(co_hbm.at[idl], xdv, gsem)

        def wait_gathers(b):
            pv, qv, xsv, xdv, gsem, _ = bufs[b]
            pltpu.make_async_copy(p_hbm.at[idx_s.at[pl.ds(0, KG)]], pv, gsem).wait()
            pltpu.make_async_copy(q_hbm.at[idx_d.at[pl.ds(0, KG)]], qv, gsem).wait()
            pltpu.make_async_copy(co_hbm.at[idx_s.at[pl.ds(0, KG)]], xsv, gsem).wait()
            pltpu.make_async_copy(co_hbm.at[idx_d.at[pl.ds(0, KG)]], xdv, gsem).wait()

        def compute(b):
            pv, qv, xsv, xdv, _, _ = bufs[b]

            def edge(e, carry):
                for j in range(8):
                    sl = pl.ds(16 * j, 16)
                    pv[e, sl] = pv[e, sl] + qv[e, sl]
                xsv[e, :] = xsv[e, :] - xdv[e, :]
                return carry

            lax.fori_loop(0, KG, edge, 0, unroll=2)

        def issue_wb(slot, b):
            pv, _, xsv, _, _, wsem = bufs[b]
            off = base + slot * KG
            pltpu.async_copy(pv, z_hbm.at[pl.ds(off, KG)], wsem)
            pltpu.async_copy(xsv, dx_hbm.at[pl.ds(off, KG)], wsem)

        def wait_wb(b):
            pv, _, xsv, _, _, wsem = bufs[b]
            pltpu.make_async_copy(pv, z_hbm.at[pl.ds(base, KG)], wsem).wait()
            pltpu.make_async_copy(xsv, dx_hbm.at[pl.ds(base, KG)], wsem).wait()

        issue_gathers(0, 0)

        def pair(i, carry):
            slot0 = 2 * i
            wait_gathers(0)

            @pl.when(i >= 1)
            def _():
                wait_wb(1)

            issue_gathers(slot0 + 1, 1)
            compute(0)
            issue_wb(slot0, 0)
            wait_gathers(1)
            wait_wb(0)
            issue_gathers(slot0 + 2, 0)
            compute(1)
            issue_wb(slot0 + 1, 1)
            return carry

        lax.fori_loop(0, (NCH - 1) // 2, pair, 0)
        wait_gathers(0)
        compute(0)
        wait_wb(1)
        issue_wb(NCH - 1, 0)
        wait_wb(0)

    return k(P, Q, copad, src, dst)


def _tc_edge(Z0, DX, w1c, W_e2, b_e2, W_c1, b_c1, W_c2, b_c2):
    BE = 3200

    def body(z_ref, dx_ref, w1c_ref, w2, b2, wc1, bc1, wc2, bc2, he_ref, xe_ref):
        dx = dx_ref[...]
        dist2 = jnp.sum(dx * dx, axis=-1, keepdims=True)
        z = z_ref[...] + dist2 * w1c_ref[...]
        a1 = _silu(z)
        he = _silu(jnp.dot(a1, w2[...], preferred_element_type=jnp.float32) + b2[...])
        t = _silu(jnp.dot(he, wc1[...], preferred_element_type=jnp.float32) + bc1[...])
        coef = jnp.dot(t, wc2[...], preferred_element_type=jnp.float32) + bc2[...]
        he_ref[...] = he
        xe_ref[...] = dx * coef

    return pl.pallas_call(
        body,
        grid=(E // BE,),
        in_specs=[
            pl.BlockSpec((BE, D), lambda i: (i, 0)),
            pl.BlockSpec((BE, 16), lambda i: (i, 0)),
            pl.BlockSpec((1, D), lambda i: (0, 0)),
            pl.BlockSpec((D, D), lambda i: (0, 0)),
            pl.BlockSpec((1, D), lambda i: (0, 0)),
            pl.BlockSpec((D, D), lambda i: (0, 0)),
            pl.BlockSpec((1, D), lambda i: (0, 0)),
            pl.BlockSpec((D, 1), lambda i: (0, 0)),
            pl.BlockSpec((1, 1), lambda i: (0, 0)),
        ],
        out_specs=[
            pl.BlockSpec((BE, D), lambda i: (i, 0)),
            pl.BlockSpec((BE, 16), lambda i: (i, 0)),
        ],
        out_shape=(
            jax.ShapeDtypeStruct((E, D), jnp.float32),
            jax.ShapeDtypeStruct((E, 16), jnp.float32),
        ),
    )(Z0, DX, w1c, W_e2, b_e2, W_c1, b_c1, W_c2, b_c2)


def _sc_scatter_h(HE, dst, zero_h):
    @functools.partial(
        pl.kernel,
        out_type=jax.ShapeDtypeStruct((NC, NP, D), jnp.float32),
        mesh=_mesh(),
        scratch_types=[
            pltpu.VMEM((KG,), jnp.int32),
            pltpu.VMEM((KG,), jnp.int32),
            pltpu.VMEM((KG, D), jnp.float32),
            pltpu.VMEM((KG, D), jnp.float32),
            pltpu.VMEM_SHARED((NP, D), jnp.float32),
            pltpu.SemaphoreType.DMA,
            pltpu.SemaphoreType.DMA,
        ],
        compiler_params=pltpu.CompilerParams(use_tc_tiling_on_sc=True),
    )
    def k(he_hbm, dst_hbm, zh_hbm, outh_hbm,
          idx_d0, idx_d1, hev0, hev1, acc_h, lsem0, lsem1):
        c = lax.axis_index("c")
        s = lax.axis_index("s")
        rs = s * RPT
        pltpu.sync_copy(zh_hbm, acc_h.at[pl.ds(rs, RPT)])

        w = c * NS + s
        base = w * EPW
        plsc.subcore_barrier()

        bufs = ((hev0, idx_d0, lsem0), (hev1, idx_d1, lsem1))

        def issue_loads(slot, b):
            hev, idx_db, lsem = bufs[b]
            off = base + slot * KG
            pltpu.async_copy(he_hbm.at[pl.ds(off, KG)], hev, lsem)
            pltpu.async_copy(dst_hbm.at[pl.ds(off, KG)], idx_db, lsem)

        def wait_loads(b):
            hev, idx_db, lsem = bufs[b]
            pltpu.make_async_copy(he_hbm.at[pl.ds(base, KG)], hev, lsem).wait()
            pltpu.make_async_copy(dst_hbm.at[pl.ds(base, KG)], idx_db, lsem).wait()

        def scatter(b):
            hev, idx_db, _ = bufs[b]
            pltpu.sync_copy(hev, acc_h.at[idx_db], add=True)

        issue_loads(0, 0)

        def pair(i, carry):
            slot0 = 2 * i
            wait_loads(0)
            issue_loads(slot0 + 1, 1)
            scatter(0)
            wait_loads(1)
            issue_loads(slot0 + 2, 0)
            scatter(1)
            return carry

        lax.fori_loop(0, (NCH - 1) // 2, pair, 0)
        wait_loads(0)
        scatter(0)

        plsc.subcore_barrier()
        pltpu.sync_copy(acc_h.at[pl.ds(rs, RPT)], outh_hbm.at[c, pl.ds(rs, RPT)])

    return k(HE, dst, zero_h)


def _sc_scatter_x(XE, dst, zero_x):
    @functools.partial(
        pl.kernel,
        out_type=jax.ShapeDtypeStruct((NC, NP, 16), jnp.float32),
        mesh=_mesh(),
        scratch_types=[
            pltpu.VMEM((KG,), jnp.int32),
            pltpu.VMEM((KG,), jnp.int32),
            pltpu.VMEM((KG, 16), jnp.float32),
            pltpu.VMEM((KG, 16), jnp.float32),
            pltpu.VMEM_SHARED((NP, 16), jnp.float32),
            pltpu.SemaphoreType.DMA,
            pltpu.SemaphoreType.DMA,
        ],
        compiler_params=pltpu.CompilerParams(use_tc_tiling_on_sc=False),
    )
    def k(xe_hbm, dst_hbm, zx_hbm, outx_hbm,
          idx0, idx1, xev0, xev1, acc_x, lsem0, lsem1):
        c = lax.axis_index("c")
        s = lax.axis_index("s")
        rs = s * RPT
        pltpu.sync_copy(zx_hbm, acc_x.at[pl.ds(rs, RPT)])
        w = c * NS + s
        base = w * EPW
        plsc.subcore_barrier()

        bufs = ((xev0, idx0, lsem0), (xev1, idx1, lsem1))

        def issue_loads(slot, b):
            xev, idx_db, lsem = bufs[b]
            off = base + slot * KG
            pltpu.async_copy(xe_hbm.at[pl.ds(off, KG)], xev, lsem)
            pltpu.async_copy(dst_hbm.at[pl.ds(off, KG)], idx_db, lsem)

        def wait_loads(b):
            xev, idx_db, lsem = bufs[b]
            pltpu.make_async_copy(xe_hbm.at[pl.ds(base, KG)], xev, lsem).wait()
            pltpu.make_async_copy(dst_hbm.at[pl.ds(base, KG)], idx_db, lsem).wait()

        def scatter(b):
            xev, idx_db, _ = bufs[b]
            pltpu.sync_copy(xev, acc_x.at[idx_db], add=True)

        issue_loads(0, 0)

        def pair(i, carry):
            slot0 = 2 * i
            wait_loads(0)
            issue_loads(slot0 + 1, 1)
            scatter(0)
            wait_loads(1)
            issue_loads(slot0 + 2, 0)
            scatter(1)
            return carry

        lax.fori_loop(0, (NCH - 1) // 2, pair, 0)
        wait_loads(0)
        scatter(0)

        plsc.subcore_barrier()
        pltpu.sync_copy(acc_x.at[pl.ds(rs, RPT)], outx_hbm.at[c, pl.ds(rs, RPT)])

    return k(XE, dst, zero_x)


def _tc_node(feat, coordinate, PH, PX, Wn1a, Wn1b, b_n1, W_n2, b_n2):
    BN = 1000

    def body(f_ref, co_ref, ph, px, wa, wb, b1, w2, b2, h_ref, x_ref):
        hagg = ph[0] + ph[1]
        a = _silu(jnp.dot(f_ref[...], wa[...], preferred_element_type=jnp.float32)
                  + jnp.dot(hagg, wb[...], preferred_element_type=jnp.float32)
                  + b1[...])
        h_ref[...] = jnp.dot(a, w2[...], preferred_element_type=jnp.float32) + b2[...]
        xagg = px[0] + px[1]
        x_ref[...] = co_ref[...] + xagg[:, :3]

    return pl.pallas_call(
        body,
        grid=(N // BN,),
        in_specs=[
            pl.BlockSpec((BN, D), lambda i: (i, 0)),
            pl.BlockSpec((BN, 3), lambda i: (i, 0)),
            pl.BlockSpec((NC, BN, D), lambda i: (0, i, 0)),
            pl.BlockSpec((NC, BN, 16), lambda i: (0, i, 0)),
            pl.BlockSpec((D, D), lambda i: (0, 0)),
            pl.BlockSpec((D, D), lambda i: (0, 0)),
            pl.BlockSpec((1, D), lambda i: (0, 0)),
            pl.BlockSpec((D, D), lambda i: (0, 0)),
            pl.BlockSpec((1, D), lambda i: (0, 0)),
        ],
        out_specs=[
            pl.BlockSpec((BN, D), lambda i: (i, 0)),
            pl.BlockSpec((BN, 3), lambda i: (i, 0)),
        ],
        out_shape=(
            jax.ShapeDtypeStruct((N, D), jnp.float32),
            jax.ShapeDtypeStruct((N, 3), jnp.float32),
        ),
    )(feat, coordinate, PH, PX, Wn1a, Wn1b, b_n1, W_n2, b_n2)


def kernel(feat, coordinate, edge_index, W_e1, b_e1, W_e2, b_e2,
           W_c1, b_c1, W_c2, b_c2, W_n1, b_n1, W_n2, b_n2):
    src = edge_index[0]
    dst = edge_index[1]
    W1a = W_e1[:D]
    W1b = W_e1[D:2 * D]
    w1c = W_e1[2 * D].reshape(1, D)
    copad = jnp.pad(coordinate, ((0, 0), (0, 13)))

    P, Q = _tc_pq(feat, W1a, W1b, b_e1.reshape(1, D))
    Z0, DX = _sc_gather_combine(P, Q, copad, src, dst)
    HE, XE = _tc_edge(Z0, DX, w1c, W_e2, b_e2.reshape(1, D),
                      W_c1, b_c1.reshape(1, D), W_c2, b_c2.reshape(1, 1))
    zero_h = jnp.zeros((RPT, D), jnp.float32)
    zero_x = jnp.zeros((RPT, 16), jnp.float32)
    PH = _sc_scatter_h(HE, dst, zero_h)
    PX = _sc_scatter_x(XE, dst, zero_x)

    h_out, x_out = _tc_node(feat, coordinate, PH, PX,
                            W_n1[:D], W_n1[D:], b_n1.reshape(1, D),
                            W_n2, b_n2.reshape(1, D))
    return (h_out, x_out)


# speedup vs baseline: 1.0930x; 1.0569x over previous
import functools

import jax
import jax.numpy as jnp
from jax import lax
from jax.experimental import pallas as pl
from jax.experimental.pallas import tpu as pltpu
from jax.experimental.pallas import tpu_sc as plsc

N = 10000
E = 320000
D = 128

NC, NS = 2, 16
NW = NC * NS
EPW = E // NW
KG = 80
NCH = EPW // KG
NP = 10240
RPT = NP // NS


def _mesh():
    return plsc.VectorSubcoreMesh(
        core_axis_name="c", subcore_axis_name="s", num_cores=NC, num_subcores=NS)


def _silu(x):
    return x * jax.nn.sigmoid(x)


def _tc_pq(feat, W1a, W1b, b_e1):
    BN = 1000

    def body(f_ref, wa, wb, b1, p_ref, q_ref):
        f = f_ref[...]
        p_ref[...] = jnp.dot(f, wa[...], preferred_element_type=jnp.float32) + b1[...]
        q_ref[...] = jnp.dot(f, wb[...], preferred_element_type=jnp.float32)

    return pl.pallas_call(
        body,
        grid=(N // BN,),
        in_specs=[
            pl.BlockSpec((BN, D), lambda i: (i, 0)),
            pl.BlockSpec((D, D), lambda i: (0, 0)),
            pl.BlockSpec((D, D), lambda i: (0, 0)),
            pl.BlockSpec((1, D), lambda i: (0, 0)),
        ],
        out_specs=[
            pl.BlockSpec((BN, D), lambda i: (i, 0)),
            pl.BlockSpec((BN, D), lambda i: (i, 0)),
        ],
        out_shape=(
            jax.ShapeDtypeStruct((N, D), jnp.float32),
            jax.ShapeDtypeStruct((N, D), jnp.float32),
        ),
    )(feat, W1a, W1b, b_e1)


def _sc_gather_combine(P, Q, copad, src, dst):
    @functools.partial(
        pl.kernel,
        out_type=(
            jax.ShapeDtypeStruct((E, D), jnp.float32),
            jax.ShapeDtypeStruct((E, 16), jnp.float32),
        ),
        mesh=_mesh(),
        scratch_types=[
            pltpu.VMEM((EPW,), jnp.int32),
            pltpu.VMEM((EPW,), jnp.int32),
            pltpu.VMEM((KG, D), jnp.float32),
            pltpu.VMEM((KG, D), jnp.float32),
            pltpu.VMEM((KG, 16), jnp.float32),
            pltpu.VMEM((KG, 16), jnp.float32),
            pltpu.VMEM((KG, D), jnp.float32),
            pltpu.VMEM((KG, D), jnp.float32),
            pltpu.VMEM((KG, 16), jnp.float32),
            pltpu.VMEM((KG, 16), jnp.float32),
            pltpu.SemaphoreType.DMA,
            pltpu.SemaphoreType.DMA,
            pltpu.SemaphoreType.DMA,
            pltpu.SemaphoreType.DMA,
        ],
        compiler_params=pltpu.CompilerParams(use_tc_tiling_on_sc=False),
    )
    def k(p_hbm, q_hbm, co_hbm, src_hbm, dst_hbm, z_hbm, dx_hbm,
          idx_s, idx_d, pv0, qv0, xsv0, xdv0, pv1, qv1, xsv1, xdv1,
          gsem0, gsem1, wsem0, wsem1):
        c = lax.axis_index("c")
        s = lax.axis_index("s")
        base = (s * NC + c) * EPW
        pltpu.sync_copy(src_hbm.at[pl.ds(base, EPW)], idx_s)
        pltpu.sync_copy(dst_hbm.at[pl.ds(base, EPW)], idx_d)

        bufs = ((pv0, qv0, xsv0, xdv0, gsem0, wsem0),
                (pv1, qv1, xsv1, xdv1, gsem1, wsem1))

        def issue_gathers(slot, b):
            pv, qv, xsv, xdv, gsem, _ = bufs[b]
            isl = idx_s.at[pl.ds(slot * KG, KG)]
            idl = idx_d.at[pl.ds(slot * KG, KG)]
            pltpu.async_copy(p_hbm.at[isl], pv, gsem)
            pltpu.async_copy(q_hbm.at[idl], qv, gsem)
            pltpu.async_copy(co_hbm.at[isl], xsv, gsem)
            pltpu.async_copy(co_hbm.at[idl], xdv, gsem)

        def wait_gathers(b):
            pv, qv, xsv, xdv, gsem, _ = bufs[b]
            pltpu.make_async_copy(p_hbm.at[idx_s.at[pl.ds(0, KG)]], pv, gsem).wait()
            pltpu.make_async_copy(q_hbm.at[idx_d.at[pl.ds(0, KG)]], qv, gsem).wait()
            pltpu.make_async_copy(co_hbm.at[idx_s.at[pl.ds(0, KG)]], xsv, gsem).wait()
            pltpu.make_async_copy(co_hbm.at[idx_d.at[pl.ds(0, KG)]], xdv, gsem).wait()

        def compute(b):
            pv, qv, xsv, xdv, _, _ = bufs[b]

            def edge(e, carry):
                for j in range(8):
                    sl = pl.ds(16 * j, 16)
                    pv[e, sl] = pv[e, sl] + qv[e, sl]
                xsv[e, :] = xsv[e, :] - xdv[e, :]
                return carry

            lax.fori_loop(0, KG, edge, 0, unroll=2)

        def issue_wb(slot, b):
            pv, _, xsv, _, _, wsem = bufs[b]
            off = base + slot * KG
            pltpu.async_copy(pv, z_hbm.at[pl.ds(off, KG)], wsem)
            pltpu.async_copy(xsv, dx_hbm.at[pl.ds(off, KG)], wsem)

        def wait_wb(b):
            pv, _, xsv, _, _, wsem = bufs[b]
            pltpu.make_async_copy(pv, z_hbm.at[pl.ds(base, KG)], wsem).wait()
            pltpu.make_async_copy(xsv, dx_hbm.at[pl.ds(base, KG)], wsem).wait()

        issue_gathers(0, 0)

        def pair(i, carry):
            slot0 = 2 * i
            wait_gathers(0)

            @pl.when(i >= 1)
            def _():
                wait_wb(1)

            issue_gathers(slot0 + 1, 1)
            compute(0)
            issue_wb(slot0, 0)
            wait_gathers(1)
            wait_wb(0)
            issue_gathers(slot0 + 2, 0)
            compute(1)
            issue_wb(slot0 + 1, 1)
            return carry

        lax.fori_loop(0, (NCH - 1) // 2, pair, 0)
        wait_gathers(0)
        compute(0)
        wait_wb(1)
        issue_wb(NCH - 1, 0)
        wait_wb(0)

    return k(P, Q, copad, src, dst)


def _tc_edge(Z0, DX, w1c, W_e2, b_e2, W_c1, b_c1, W_c2, b_c2):
    BE = 3200

    def body(z_ref, dx_ref, w1c_ref, w2, b2, wc1, bc1, wc2, bc2, he_ref, xe_ref):
        dx = dx_ref[...]
        dist2 = jnp.sum(dx * dx, axis=-1, keepdims=True)
        z = z_ref[...] + dist2 * w1c_ref[...]
        a1 = _silu(z)
        he = _silu(jnp.dot(a1, w2[...], preferred_element_type=jnp.float32) + b2[...])
        t = _silu(jnp.dot(he, wc1[...], preferred_element_type=jnp.float32) + bc1[...])
        coef = jnp.dot(t, wc2[...], preferred_element_type=jnp.float32) + bc2[...]
        he_ref[...] = he
        xe_ref[...] = dx * coef

    return pl.pallas_call(
        body,
        grid=(E // BE,),
        in_specs=[
            pl.BlockSpec((BE, D), lambda i: (i, 0)),
            pl.BlockSpec((BE, 16), lambda i: (i, 0)),
            pl.BlockSpec((1, D), lambda i: (0, 0)),
            pl.BlockSpec((D, D), lambda i: (0, 0)),
            pl.BlockSpec((1, D), lambda i: (0, 0)),
            pl.BlockSpec((D, D), lambda i: (0, 0)),
            pl.BlockSpec((1, D), lambda i: (0, 0)),
            pl.BlockSpec((D, 1), lambda i: (0, 0)),
            pl.BlockSpec((1, 1), lambda i: (0, 0)),
        ],
        out_specs=[
            pl.BlockSpec((BE, D), lambda i: (i, 0)),
            pl.BlockSpec((BE, 16), lambda i: (i, 0)),
        ],
        out_shape=(
            jax.ShapeDtypeStruct((E, D), jnp.float32),
            jax.ShapeDtypeStruct((E, 16), jnp.float32),
        ),
    )(Z0, DX, w1c, W_e2, b_e2, W_c1, b_c1, W_c2, b_c2)


def _sc_scatter(HE, XE, dst, zero_h, zero_x):
    @functools.partial(
        pl.kernel,
        out_type=(
            jax.ShapeDtypeStruct((NC, NP, D), jnp.float32),
            jax.ShapeDtypeStruct((NC, NP, 16), jnp.float32),
        ),
        mesh=_mesh(),
        scratch_types=[
            pltpu.VMEM((KG,), jnp.int32),
            pltpu.VMEM((KG,), jnp.int32),
            pltpu.VMEM((KG, D), jnp.float32),
            pltpu.VMEM((KG, D), jnp.float32),
            pltpu.VMEM((KG, 16), jnp.float32),
            pltpu.VMEM((KG, 16), jnp.float32),
            pltpu.VMEM_SHARED((NP, D), jnp.float32),
            pltpu.VMEM_SHARED((NP, 16), jnp.float32),
            pltpu.SemaphoreType.DMA,
            pltpu.SemaphoreType.DMA,
        ],
        compiler_params=pltpu.CompilerParams(use_tc_tiling_on_sc=False),
    )
    def k(he_hbm, xe_hbm, dst_hbm, zh_hbm, zx_hbm, outh_hbm, outx_hbm,
          idx0, idx1, hev0, hev1, xev0, xev1, acc_h, acc_x, lsem0, lsem1):
        c = lax.axis_index("c")
        s = lax.axis_index("s")
        rs = s * RPT
        pltpu.sync_copy(zh_hbm, acc_h.at[pl.ds(rs, RPT)])
        pltpu.sync_copy(zx_hbm, acc_x.at[pl.ds(rs, RPT)])
        w = c * NS + s
        base = w * EPW
        plsc.subcore_barrier()

        bufs = ((hev0, xev0, idx0, lsem0), (hev1, xev1, idx1, lsem1))

        def issue_loads(slot, b):
            hev, xev, idx_db, lsem = bufs[b]
            off = base + slot * KG
            pltpu.async_copy(he_hbm.at[pl.ds(off, KG)], hev, lsem)
            pltpu.async_copy(xe_hbm.at[pl.ds(off, KG)], xev, lsem)
            pltpu.async_copy(dst_hbm.at[pl.ds(off, KG)], idx_db, lsem)

        def wait_loads(b):
            hev, xev, idx_db, lsem = bufs[b]
            pltpu.make_async_copy(he_hbm.at[pl.ds(base, KG)], hev, lsem).wait()
            pltpu.make_async_copy(xe_hbm.at[pl.ds(base, KG)], xev, lsem).wait()
            pltpu.make_async_copy(dst_hbm.at[pl.ds(base, KG)], idx_db, lsem).wait()

        def scatter(b):
            hev, xev, idx_db, _ = bufs[b]
            pltpu.sync_copy(hev, acc_h.at[idx_db], add=True)
            pltpu.sync_copy(xev, acc_x.at[idx_db], add=True)

        issue_loads(0, 0)

        def pair(i, carry):
            slot0 = 2 * i
            wait_loads(0)
            issue_loads(slot0 + 1, 1)
            scatter(0)
            wait_loads(1)
            issue_loads(slot0 + 2, 0)
            scatter(1)
            return carry

        lax.fori_loop(0, (NCH - 1) // 2, pair, 0)
        wait_loads(0)
        scatter(0)

        plsc.subcore_barrier()
        pltpu.sync_copy(acc_h.at[pl.ds(rs, RPT)], outh_hbm.at[c, pl.ds(rs, RPT)])
        pltpu.sync_copy(acc_x.at[pl.ds(rs, RPT)], outx_hbm.at[c, pl.ds(rs, RPT)])

    return k(HE, XE, dst, zero_h, zero_x)


def _tc_node(feat, coordinate, PH, PX, Wn1a, Wn1b, b_n1, W_n2, b_n2):
    BN = 1000

    def body(f_ref, co_ref, ph, px, wa, wb, b1, w2, b2, h_ref, x_ref):
        hagg = ph[0] + ph[1]
        a = _silu(jnp.dot(f_ref[...], wa[...], preferred_element_type=jnp.float32)
                  + jnp.dot(hagg, wb[...], preferred_element_type=jnp.float32)
                  + b1[...])
        h_ref[...] = jnp.dot(a, w2[...], preferred_element_type=jnp.float32) + b2[...]
        xagg = px[0] + px[1]
        x_ref[...] = co_ref[...] + xagg[:, :3]

    return pl.pallas_call(
        body,
        grid=(N // BN,),
        in_specs=[
            pl.BlockSpec((BN, D), lambda i: (i, 0)),
            pl.BlockSpec((BN, 3), lambda i: (i, 0)),
            pl.BlockSpec((NC, BN, D), lambda i: (0, i, 0)),
            pl.BlockSpec((NC, BN, 16), lambda i: (0, i, 0)),
            pl.BlockSpec((D, D), lambda i: (0, 0)),
            pl.BlockSpec((D, D), lambda i: (0, 0)),
            pl.BlockSpec((1, D), lambda i: (0, 0)),
            pl.BlockSpec((D, D), lambda i: (0, 0)),
            pl.BlockSpec((1, D), lambda i: (0, 0)),
        ],
        out_specs=[
            pl.BlockSpec((BN, D), lambda i: (i, 0)),
            pl.BlockSpec((BN, 3), lambda i: (i, 0)),
        ],
        out_shape=(
            jax.ShapeDtypeStruct((N, D), jnp.float32),
            jax.ShapeDtypeStruct((N, 3), jnp.float32),
        ),
    )(feat, coordinate, PH, PX, Wn1a, Wn1b, b_n1, W_n2, b_n2)


def kernel(feat, coordinate, edge_index, W_e1, b_e1, W_e2, b_e2,
           W_c1, b_c1, W_c2, b_c2, W_n1, b_n1, W_n2, b_n2):
    src = edge_index[0]
    dst = edge_index[1]
    W1a = W_e1[:D]
    W1b = W_e1[D:2 * D]
    w1c = W_e1[2 * D].reshape(1, D)
    copad = jnp.pad(coordinate, ((0, 0), (0, 13)))

    P, Q = _tc_pq(feat, W1a, W1b, b_e1.reshape(1, D))
    Z0, DX = _sc_gather_combine(P, Q, copad, src, dst)
    HE, XE = _tc_edge(Z0, DX, w1c, W_e2, b_e2.reshape(1, D),
                      W_c1, b_c1.reshape(1, D), W_c2, b_c2.reshape(1, 1))
    zero_h = jnp.zeros((RPT, D), jnp.float32)
    zero_x = jnp.zeros((RPT, 16), jnp.float32)
    PH, PX = _sc_scatter(HE, XE, dst, zero_h, zero_x)

    h_out, x_out = _tc_node(feat, coordinate, PH, PX,
                            W_n1[:D], W_n1[D:], b_n1.reshape(1, D),
                            W_n2, b_n2.reshape(1, D))
    return (h_out, x_out)
